# baseline (speedup 1.0000x reference)
import dataclasses
import functools

import jax
import jax.numpy as jnp
from jax import lax
from jax.experimental import pallas as pl
from jax.experimental.pallas import tpu as pltpu
from jax.experimental.pallas import tpu_sc as plsc

_RATE = 0.1

_L = 16
_NC = 2
_NS = 16
_NW = _NC * _NS
_N = 1_000_000
_ROWS = _N // _L
_RPW = (_ROWS // _NW) // 8 * 8
_MAIN = _RPW * _NW
_REM = _ROWS - _MAIN
_NT = 128
_SUB = 4
_CR = _RPW // _SUB

def _apply_adds(buf, tgt_v, stim_v, tmod_v, base, nrows):
    gate_zero = tmod_v[...] == 0.0
    for j in range(_NT // _L):
        t = tgt_v[pl.ds(j * _L, _L)]
        s = stim_v[pl.ds(j * _L, _L)]
        s = jnp.where(gate_zero, s, jnp.zeros_like(s))
        row = lax.div(t, _L) - base
        col = lax.rem(t, _L)
        inb = (row >= 0) & (row < nrows)
        row_c = jnp.minimum(jnp.maximum(row, 0), nrows - 1)
        plsc.addupdate_scatter(buf, [row_c, col], s, mask=inb)


@functools.lru_cache(maxsize=1)
def _build_stim_kernel():
    mesh = plsc.VectorSubcoreMesh(
        core_axis_name="c", subcore_axis_name="s",
        num_cores=_NC, num_subcores=_NS,
    )
    cp = pltpu.CompilerParams()
    if "needs_layout_passes" in pltpu.CompilerParams.__dataclass_fields__:
        cp = dataclasses.replace(cp, needs_layout_passes=False)
    if "use_tc_tiling_on_sc" in pltpu.CompilerParams.__dataclass_fields__:
        cp = dataclasses.replace(cp, use_tc_tiling_on_sc=False)

    @functools.partial(
        pl.kernel,
        compiler_params=cp,
        out_type=jax.ShapeDtypeStruct((_ROWS, _L), jnp.float32),
        mesh=mesh,
        scratch_types=[
            pltpu.VMEM((_SUB, _CR, _L), jnp.float32),
            pltpu.VMEM((_REM, _L), jnp.float32),
            pltpu.VMEM((_NT,), jnp.int32),
            pltpu.VMEM((_NT,), jnp.float32),
            pltpu.VMEM((_L,), jnp.float32),
            [pltpu.SemaphoreType.DMA] * _SUB,
            pltpu.SemaphoreType.DMA,
            pltpu.SemaphoreType.DMA,
        ],
    )
    def _stim_kernel(x_hbm, tgt_hbm, stim_hbm, tmod_hbm, o_hbm,
                     bufs, rbuf, tgt_v, stim_v, tmod_v, insems, outsem, tsem):
        wid = lax.axis_index("s") * _NC + lax.axis_index("c")
        base = wid * _RPW
        ins = [
            pltpu.async_copy(
                x_hbm.at[pl.ds(base + k * _CR, _CR)], bufs.at[k], insems[k])
            for k in range(_SUB)
        ]
        tin = pltpu.async_copy(x_hbm.at[pl.ds(_MAIN, _REM)], rbuf, tsem)
        pltpu.sync_copy(tgt_hbm, tgt_v)
        pltpu.sync_copy(stim_hbm, stim_v)
        pltpu.sync_copy(tmod_hbm, tmod_v)
        outs = []
        for k in range(_SUB):
            ins[k].wait()
            _apply_adds(bufs.at[k], tgt_v, stim_v, tmod_v, base + k * _CR, _CR)
            outs.append(pltpu.async_copy(
                bufs.at[k], o_hbm.at[pl.ds(base + k * _CR, _CR)], outsem))
        tin.wait()
        _apply_adds(rbuf, tgt_v, stim_v, tmod_v, _MAIN, _REM)

        @pl.when(wid == _NW - 1)
        def _tail_out():
            pltpu.sync_copy(rbuf, o_hbm.at[pl.ds(_MAIN, _REM)])

        for o in outs:
            o.wait()

    return _stim_kernel


def kernel(t, out, targets, stimulation_strength):
    tmod = (t % (1.0 / _RATE)).astype(jnp.float32)
    tmod_vec = jnp.broadcast_to(tmod, (_L,))
    x2d = out.reshape(_ROWS, _L)
    tgt = targets.astype(jnp.int32)
    o2d = _build_stim_kernel()(x2d, tgt, stimulation_strength, tmod_vec)
    return o2d.reshape(_N)


# speedup vs baseline: 1.0454x; 1.0454x over previous
import dataclasses
import functools

import jax
import jax.numpy as jnp
from jax import lax
from jax.experimental import pallas as pl
from jax.experimental.pallas import tpu as pltpu
from jax.experimental.pallas import tpu_sc as plsc

_RATE = 0.1

_L = 16
_NC = 2
_NS = 16
_N = 1_000_000
_ROWS = _N // _L
_NT = 128

_HA = 2264
_LA = 1640
_TAIL = _ROWS - _NS * (_HA + _LA)
_LMAX = _LA + _TAIL
_HEAVY_CORE = 1


def _apply_adds(buf, tgt_v, stim_v, tmod_v, base, nrows):
    gate_zero = tmod_v[...] == 0.0
    for j in range(_NT // _L):
        t = tgt_v[pl.ds(j * _L, _L)]
        s = stim_v[pl.ds(j * _L, _L)]
        s = jnp.where(gate_zero, s, jnp.zeros_like(s))
        row = lax.div(t, _L) - base
        col = lax.rem(t, _L)
        inb = (row >= 0) & (row < nrows)
        row_c = jnp.minimum(jnp.maximum(row, 0), nrows - 1)
        plsc.addupdate_scatter(buf, [row_c, col], s, mask=inb)


@functools.lru_cache(maxsize=1)
def _build_stim_kernel():
    mesh = plsc.VectorSubcoreMesh(
        core_axis_name="c", subcore_axis_name="s",
        num_cores=_NC, num_subcores=_NS,
    )
    cp = pltpu.CompilerParams()
    if "needs_layout_passes" in pltpu.CompilerParams.__dataclass_fields__:
        cp = dataclasses.replace(cp, needs_layout_passes=False)
    if "use_tc_tiling_on_sc" in pltpu.CompilerParams.__dataclass_fields__:
        cp = dataclasses.replace(cp, use_tc_tiling_on_sc=False)

    @functools.partial(
        pl.kernel,
        compiler_params=cp,
        out_type=jax.ShapeDtypeStruct((_ROWS, _L), jnp.float32),
        mesh=mesh,
        scratch_types=[
            pltpu.VMEM((_LMAX if _LMAX > _HA else _HA, _L), jnp.float32),
            pltpu.VMEM((_NT,), jnp.int32),
            pltpu.VMEM((_NT,), jnp.float32),
            pltpu.VMEM((_L,), jnp.float32),
            pltpu.SemaphoreType.DMA,
        ],
    )
    def _stim_kernel(x_hbm, tgt_hbm, stim_hbm, tmod_hbm, o_hbm,
                     buf, tgt_v, stim_v, tmod_v, sem):
        c = lax.axis_index("c")
        s = lax.axis_index("s")

        def do_chunk(base, nrows):
            bslc = buf.at[pl.ds(0, nrows)]
            cin = pltpu.async_copy(x_hbm.at[pl.ds(base, nrows)], bslc, sem)
            pltpu.sync_copy(tgt_hbm, tgt_v)
            pltpu.sync_copy(stim_hbm, stim_v)
            pltpu.sync_copy(tmod_hbm, tmod_v)
            cin.wait()
            _apply_adds(bslc, tgt_v, stim_v, tmod_v, base, nrows)
            pltpu.async_copy(bslc, o_hbm.at[pl.ds(base, nrows)], sem).wait()

        @pl.when(c == _HEAVY_CORE)
        def _heavy():
            do_chunk(s * _HA, _HA)

        @pl.when((c != _HEAVY_CORE) & (s < _NS - 1))
        def _light():
            do_chunk(_NS * _HA + s * _LA, _LA)

        @pl.when((c != _HEAVY_CORE) & (s == _NS - 1))
        def _light_tail():
            do_chunk(_NS * _HA + (_NS - 1) * _LA, _LMAX)

    return _stim_kernel


def kernel(t, out, targets, stimulation_strength):
    tmod = (t % (1.0 / _RATE)).astype(jnp.float32)
    tmod_vec = jnp.broadcast_to(tmod, (_L,))
    x2d = out.reshape(_ROWS, _L)
    tgt = targets.astype(jnp.int32)
    o2d = _build_stim_kernel()(x2d, tgt, stimulation_strength, tmod_vec)
    return o2d.reshape(_N)


# speedup vs baseline: 1.0529x; 1.0071x over previous
import dataclasses
import functools

import jax
import jax.numpy as jnp
from jax import lax
from jax.experimental import pallas as pl
from jax.experimental.pallas import tpu as pltpu
from jax.experimental.pallas import tpu_sc as plsc

_RATE = 0.1

_L = 16
_NC = 2
_NS = 16
_N = 1_000_000
_ROWS = _N // _L
_NT = 128

_HA = 2264
_LA = 1640
_TAIL = _ROWS - _NS * (_HA + _LA)
_LMAX = _LA + _TAIL
_HEAVY_CORE = 1


def _apply_adds(buf, tgt_v, stim_v, gate_zero, base, nrows):
    for j in range(_NT // _L):
        t = tgt_v[pl.ds(j * _L, _L)]
        s = stim_v[pl.ds(j * _L, _L)]
        s = jnp.where(gate_zero, s, jnp.zeros_like(s))
        row = lax.div(t, _L) - base
        col = lax.rem(t, _L)
        inb = (row >= 0) & (row < nrows)
        row_c = jnp.minimum(jnp.maximum(row, 0), nrows - 1)
        plsc.addupdate_scatter(buf, [row_c, col], s, mask=inb)


@functools.lru_cache(maxsize=1)
def _build_stim_kernel():
    mesh = plsc.VectorSubcoreMesh(
        core_axis_name="c", subcore_axis_name="s",
        num_cores=_NC, num_subcores=_NS,
    )
    cp = pltpu.CompilerParams()
    if "needs_layout_passes" in pltpu.CompilerParams.__dataclass_fields__:
        cp = dataclasses.replace(cp, needs_layout_passes=False)
    if "use_tc_tiling_on_sc" in pltpu.CompilerParams.__dataclass_fields__:
        cp = dataclasses.replace(cp, use_tc_tiling_on_sc=False)
    if "skip_device_barrier" in pltpu.CompilerParams.__dataclass_fields__:
        cp = dataclasses.replace(cp, skip_device_barrier=True)

    @functools.partial(
        pl.kernel,
        compiler_params=cp,
        out_type=jax.ShapeDtypeStruct((_ROWS, _L), jnp.float32),
        mesh=mesh,
        scratch_types=[
            pltpu.VMEM((_LMAX if _LMAX > _HA else _HA, _L), jnp.float32),
            pltpu.VMEM((_NT,), jnp.int32),
            pltpu.VMEM((_NT,), jnp.float32),
            pltpu.VMEM((_L,), jnp.float32),
            pltpu.SemaphoreType.DMA,
        ],
    )
    def _stim_kernel(x_hbm, tgt_hbm, stim_hbm, tmod_hbm, o_hbm,
                     buf, tgt_v, stim_v, tmod_v, sem):
        c = lax.axis_index("c")
        s = lax.axis_index("s")

        def do_chunk(base, nrows):
            bslc = buf.at[pl.ds(0, nrows)]
            cin = pltpu.async_copy(x_hbm.at[pl.ds(base, nrows)], bslc, sem)
            pltpu.sync_copy(tgt_hbm, tgt_v)
            pltpu.sync_copy(stim_hbm, stim_v)
            pltpu.sync_copy(tmod_hbm, tmod_v)
            gate_zero = tmod_v[...] == 0.0
            cin.wait()
            _apply_adds(bslc, tgt_v, stim_v, gate_zero, base, nrows)
            pltpu.async_copy(bslc, o_hbm.at[pl.ds(base, nrows)], sem).wait()

        @pl.when(c == _HEAVY_CORE)
        def _heavy():
            do_chunk(s * _HA, _HA)

        @pl.when((c != _HEAVY_CORE) & (s < _NS - 1))
        def _light():
            do_chunk(_NS * _HA + s * _LA, _LA)

        @pl.when((c != _HEAVY_CORE) & (s == _NS - 1))
        def _light_tail():
            do_chunk(_NS * _HA + (_NS - 1) * _LA, _LMAX)

    return _stim_kernel


def kernel(t, out, targets, stimulation_strength):
    tmod = (t % (1.0 / _RATE)).astype(jnp.float32)
    tmod_vec = jnp.broadcast_to(tmod, (_L,))
    x2d = out.reshape(_ROWS, _L)
    tgt = targets.astype(jnp.int32)
    o2d = _build_stim_kernel()(x2d, tgt, stimulation_strength, tmod_vec)
    return o2d.reshape(_N)


# speedup vs baseline: 1.0531x; 1.0002x over previous
import dataclasses
import functools

import jax
import jax.numpy as jnp
from jax import lax
from jax.experimental import pallas as pl
from jax.experimental.pallas import tpu as pltpu
from jax.experimental.pallas import tpu_sc as plsc

_RATE = 0.1

_L = 16
_NC = 2
_NS = 16
_N = 1_000_000
_NT = 128

_HE = 2264 * _L
_LE = 1640 * _L
_TAIL = _N - _NS * (_HE + _LE)
_LMAX = _LE + _TAIL
_HEAVY_CORE = 1


def _apply_adds(buf, tgt_v, stim_v, gate_zero, base, nelems):
    for j in range(_NT // _L):
        t = tgt_v[pl.ds(j * _L, _L)]
        s = stim_v[pl.ds(j * _L, _L)]
        s = jnp.where(gate_zero, s, jnp.zeros_like(s))
        et = t - base
        inb = (et >= 0) & (et < nelems)
        et_c = jnp.minimum(jnp.maximum(et, 0), nelems - 1)
        plsc.addupdate_scatter(buf, [et_c], s, mask=inb)


@functools.lru_cache(maxsize=1)
def _build_stim_kernel():
    mesh = plsc.VectorSubcoreMesh(
        core_axis_name="c", subcore_axis_name="s",
        num_cores=_NC, num_subcores=_NS,
    )
    cp = pltpu.CompilerParams()
    if "needs_layout_passes" in pltpu.CompilerParams.__dataclass_fields__:
        cp = dataclasses.replace(cp, needs_layout_passes=False)
    if "use_tc_tiling_on_sc" in pltpu.CompilerParams.__dataclass_fields__:
        cp = dataclasses.replace(cp, use_tc_tiling_on_sc=False)
    if "skip_device_barrier" in pltpu.CompilerParams.__dataclass_fields__:
        cp = dataclasses.replace(cp, skip_device_barrier=True)

    @functools.partial(
        pl.kernel,
        compiler_params=cp,
        out_type=jax.ShapeDtypeStruct((_N,), jnp.float32),
        mesh=mesh,
        scratch_types=[
            pltpu.VMEM((_LMAX if _LMAX > _HE else _HE,), jnp.float32),
            pltpu.VMEM((_NT,), jnp.int32),
            pltpu.VMEM((_NT,), jnp.float32),
            pltpu.VMEM((_L,), jnp.float32),
            pltpu.SemaphoreType.DMA,
        ],
    )
    def _stim_kernel(x_hbm, tgt_hbm, stim_hbm, tmod_hbm, o_hbm,
                     buf, tgt_v, stim_v, tmod_v, sem):
        c = lax.axis_index("c")
        s = lax.axis_index("s")

        def do_chunk(base, nelems):
            bslc = buf.at[pl.ds(0, nelems)]
            cin = pltpu.async_copy(x_hbm.at[pl.ds(base, nelems)], bslc, sem)
            pltpu.sync_copy(tgt_hbm, tgt_v)
            pltpu.sync_copy(stim_hbm, stim_v)
            pltpu.sync_copy(tmod_hbm, tmod_v)
            gate_zero = tmod_v[...] == 0.0
            cin.wait()
            _apply_adds(bslc, tgt_v, stim_v, gate_zero, base, nelems)
            pltpu.async_copy(bslc, o_hbm.at[pl.ds(base, nelems)], sem).wait()

        @pl.when(c == _HEAVY_CORE)
        def _heavy():
            do_chunk(s * _HE, _HE)

        @pl.when((c != _HEAVY_CORE) & (s < _NS - 1))
        def _light():
            do_chunk(_NS * _HE + s * _LE, _LE)

        @pl.when((c != _HEAVY_CORE) & (s == _NS - 1))
        def _light_tail():
            do_chunk(_NS * _HE + (_NS - 1) * _LE, _LMAX)

    return _stim_kernel


def kernel(t, out, targets, stimulation_strength):
    tmod = (t % (1.0 / _RATE)).astype(jnp.float32)
    tmod_vec = jnp.broadcast_to(tmod, (_L,))
    tgt = targets.astype(jnp.int32)
    return _build_stim_kernel()(out, tgt, stimulation_strength, tmod_vec)
